# baseline (speedup 1.0000x reference)
import functools

import jax
import jax.numpy as jnp
from jax import lax
from jax.experimental import pallas as pl
from jax.experimental.pallas import tpu as pltpu
from jax.experimental.pallas import tpu_sc as plsc

VOCAB = 100000
D = 128
BATCH = 4096
CTX = 20

NC = 2
NS = 16
NW = NC * NS

B_PER_W = BATCH // NW
CHUNK_B = 4
N_CHUNKS = B_PER_W // CHUNK_B
IDX_PER_CHUNK = CHUNK_B * CTX


def _gather_mean_body(x_hbm, emb_hbm, out_hbm, idx_v, rows_v, hacc_v, sem):
    wid = lax.axis_index("s") * NC + lax.axis_index("c")
    pltpu.sync_copy(x_hbm.at[wid], idx_v)

    def chunk_body(cc, carry):
        pltpu.async_copy(emb_hbm.at[idx_v.at[cc]], rows_v, sem).wait()
        for r in range(CHUNK_B):
            base = r * CTX
            for c in range(D // 16):
                acc = rows_v[base, pl.ds(c * 16, 16)]
                for j in range(1, CTX):
                    acc = acc + rows_v[base + j, pl.ds(c * 16, 16)]
                hacc_v[cc * CHUNK_B + r, pl.ds(c * 16, 16)] = acc * (1.0 / CTX)
        return carry

    lax.fori_loop(0, N_CHUNKS, chunk_body, 0)

    pltpu.sync_copy(hacc_v, out_hbm.at[pl.ds(wid * B_PER_W, B_PER_W)])


@functools.partial(jax.jit, static_argnames=())
def _gather_mean(x_grouped, emb):
    mesh = plsc.VectorSubcoreMesh(core_axis_name="c", subcore_axis_name="s")
    return pl.kernel(
        _gather_mean_body,
        out_type=jax.ShapeDtypeStruct((BATCH, D), jnp.float32),
        mesh=mesh,
        scratch_types=[
            pltpu.VMEM((N_CHUNKS, IDX_PER_CHUNK), jnp.int32),
            pltpu.VMEM((IDX_PER_CHUNK, D), jnp.float32),
            pltpu.VMEM((B_PER_W, D), jnp.float32),
            pltpu.SemaphoreType.DMA,
        ],
    )(x_grouped, emb)


BN = 512


def _mm_body(h_ref, w_ref, b_ref, out_ref):
    out_ref[...] = lax.dot_general(
        h_ref[...], w_ref[...],
        (((1,), (1,)), ((), ())),
        preferred_element_type=jnp.float32,
    ) + b_ref[...]


def _matmul(h, W, b2d):
    grid = (pl.cdiv(VOCAB, BN),)
    return pl.pallas_call(
        _mm_body,
        grid=grid,
        in_specs=[
            pl.BlockSpec((BATCH, D), lambda n: (0, 0)),
            pl.BlockSpec((BN, D), lambda n: (n, 0)),
            pl.BlockSpec((1, BN), lambda n: (0, n)),
        ],
        out_specs=pl.BlockSpec((BATCH, BN), lambda n: (0, n)),
        out_shape=jax.ShapeDtypeStruct((BATCH, VOCAB), jnp.float32),
    )(h, W, b2d)


def kernel(x, emb, W, b):
    x_grouped = x.astype(jnp.int32).reshape(NW, N_CHUNKS, IDX_PER_CHUNK)
    h = _gather_mean(x_grouped, emb)
    return _matmul(h, W, b.reshape(1, VOCAB))
